# baseline (speedup 1.0000x reference)
import functools

import jax
import jax.numpy as jnp
from jax import lax
from jax.experimental import pallas as pl
from jax.experimental.pallas import tpu as pltpu
from jax.experimental.pallas import tpu_sc as plsc

BATCH = 16384
D = 64
NC = 2
NS = 16
NW = NC * NS
BPW = BATCH // NW
CHUNK = 128
NCHUNK = BPW // CHUNK


def _mf_body(users_hbm, items_hbm, uf_hbm, if_hbm, out_hbm,
             uidx, iidx, urows, irows, outv, sem):
    wid = lax.axis_index("s") * NC + lax.axis_index("c")
    base = wid * BPW

    pltpu.sync_copy(users_hbm.at[wid], uidx)
    pltpu.sync_copy(items_hbm.at[wid], iidx)

    def fire(c, carry):
        for j in range(NCHUNK):
            pltpu.async_copy(
                uf_hbm.at[c].at[uidx.at[j]],
                urows.at[c, pl.ds(j * CHUNK, CHUNK)], sem)
            pltpu.async_copy(
                if_hbm.at[c].at[iidx.at[j]],
                irows.at[c, pl.ds(j * CHUNK, CHUNK)], sem)
        return carry

    lax.fori_loop(0, D, fire, 0)

    pltpu.make_async_copy(
        uf_hbm.at[pl.ds(0, D), pl.ds(0, BPW)], urows, sem).wait()
    pltpu.make_async_copy(
        if_hbm.at[pl.ds(0, D), pl.ds(0, BPW)], irows, sem).wait()

    def group(m, carry):
        s = m * 16
        accs = [jnp.zeros((16,), jnp.float32) for _ in range(4)]
        for c in range(D):
            accs[c % 4] = accs[c % 4] + (
                urows[c, pl.ds(s, 16)] * irows[c, pl.ds(s, 16)])
        outv[pl.ds(s, 16)] = (accs[0] + accs[1]) + (accs[2] + accs[3])
        return carry

    lax.fori_loop(0, BPW // 16, group, 0)

    pltpu.sync_copy(outv, out_hbm.at[pl.ds(base, BPW)])


@jax.jit
def _mf(users3d, items3d, uf_t, if_t):
    mesh = plsc.VectorSubcoreMesh(core_axis_name="c", subcore_axis_name="s")
    kern = functools.partial(
        pl.kernel,
        mesh=mesh,
        compiler_params=pltpu.CompilerParams(
            needs_layout_passes=False, use_tc_tiling_on_sc=False),
        out_type=jax.ShapeDtypeStruct((BATCH,), jnp.float32),
        scratch_types=[
            pltpu.VMEM((NCHUNK, CHUNK), jnp.int32),
            pltpu.VMEM((NCHUNK, CHUNK), jnp.int32),
            pltpu.VMEM((D, BPW), jnp.float32),
            pltpu.VMEM((D, BPW), jnp.float32),
            pltpu.VMEM((BPW,), jnp.float32),
            pltpu.SemaphoreType.DMA,
        ],
    )(_mf_body)
    return kern(users3d, items3d, uf_t, if_t)


def kernel(data, user_factors, item_factors):
    users3d = data[:, 0].astype(jnp.int32).reshape(NW, NCHUNK, CHUNK)
    items3d = data[:, 1].astype(jnp.int32).reshape(NW, NCHUNK, CHUNK)
    return _mf(users3d, items3d, user_factors.T, item_factors.T)


# speedup vs baseline: 8.6635x; 8.6635x over previous
import functools

import jax
import jax.numpy as jnp
from jax import lax
from jax.experimental import pallas as pl
from jax.experimental.pallas import tpu as pltpu
from jax.experimental.pallas import tpu_sc as plsc

BATCH = 16384
D = 64
W = 2 * D
NC = 2
NS = 16
NW = NC * NS
BPW = BATCH // NW
HALF = BPW // 2
CHUNK = 128
NCHUNK = HALF // CHUNK


def _mf_body(uq_hbm, iq_hbm, uh_hbm, ih_hbm, uf_hbm, if_hbm, out_hbm,
             uidx, iidx, uoff, ioff, urows, irows, outv, sem):
    wid = lax.axis_index("s") * NC + lax.axis_index("c")
    lane = lax.iota(jnp.int32, 16)

    for half in range(2):
        hid = wid * 2 + half
        pltpu.sync_copy(uq_hbm.at[hid], uidx)
        pltpu.sync_copy(iq_hbm.at[hid], iidx)
        pltpu.sync_copy(uh_hbm.at[hid], uoff)
        pltpu.sync_copy(ih_hbm.at[hid], ioff)

        for j in range(NCHUNK):
            pltpu.async_copy(
                uf_hbm.at[uidx.at[j]],
                urows.at[pl.ds(j * CHUNK, CHUNK)], sem)
            pltpu.async_copy(
                if_hbm.at[iidx.at[j]],
                irows.at[pl.ds(j * CHUNK, CHUNK)], sem)

        pltpu.make_async_copy(
            uf_hbm.at[pl.ds(0, HALF)], urows, sem).wait()
        pltpu.make_async_copy(
            if_hbm.at[pl.ds(0, HALF)], irows, sem).wait()

        def group(g, carry):
            s = g * 16
            rows = s + lane
            cu = uoff[pl.ds(s, 16)]
            ci = ioff[pl.ds(s, 16)]
            accs = [jnp.zeros((16,), jnp.float32) for _ in range(4)]
            for c in range(D):
                u = plsc.load_gather(urows, [rows, cu])
                v = plsc.load_gather(irows, [rows, ci])
                accs[c % 4] = accs[c % 4] + u * v
                cu = cu + 1
                ci = ci + 1
            outv[pl.ds(s, 16)] = (accs[0] + accs[1]) + (accs[2] + accs[3])
            return carry

        lax.fori_loop(0, HALF // 16, group, 0)

        pltpu.sync_copy(outv, out_hbm.at[pl.ds(hid * HALF, HALF)])


@jax.jit
def _mf(uq3d, iq3d, uh2d, ih2d, uf2, if2):
    mesh = plsc.VectorSubcoreMesh(core_axis_name="c", subcore_axis_name="s")
    kern = functools.partial(
        pl.kernel,
        mesh=mesh,
        compiler_params=pltpu.CompilerParams(
            needs_layout_passes=False, use_tc_tiling_on_sc=True),
        out_type=jax.ShapeDtypeStruct((BATCH,), jnp.float32),
        scratch_types=[
            pltpu.VMEM((NCHUNK, CHUNK), jnp.int32),
            pltpu.VMEM((NCHUNK, CHUNK), jnp.int32),
            pltpu.VMEM((HALF,), jnp.int32),
            pltpu.VMEM((HALF,), jnp.int32),
            pltpu.VMEM((HALF, W), jnp.float32),
            pltpu.VMEM((HALF, W), jnp.float32),
            pltpu.VMEM((HALF,), jnp.float32),
            pltpu.SemaphoreType.DMA,
        ],
    )(_mf_body)
    return kern(uq3d, iq3d, uh2d, ih2d, uf2, if2)


def kernel(data, user_factors, item_factors):
    users = data[:, 0].astype(jnp.int32)
    items = data[:, 1].astype(jnp.int32)
    uq3d = (users >> 1).reshape(NW * 2, NCHUNK, CHUNK)
    iq3d = (items >> 1).reshape(NW * 2, NCHUNK, CHUNK)
    uh2d = ((users & 1) * D).reshape(NW * 2, HALF)
    ih2d = ((items & 1) * D).reshape(NW * 2, HALF)
    uf2 = user_factors.reshape(500000, W)
    if2 = item_factors.reshape(500000, W)
    return _mf(uq3d, iq3d, uh2d, ih2d, uf2, if2)
